# baseline (speedup 1.0000x reference)
import jax
import jax.numpy as jnp
from jax import lax
from jax.experimental import pallas as pl
from jax.experimental.pallas import tpu as pltpu
from jax.experimental.pallas import tpu_sc as plsc

B, T, D = 16, 2048, 256
S = 256
WB = 3
L = 16
CHUNK = 32
NCH = S // CHUNK
GCH = 8
NGC = S // GCH


def _sc_body(rnn_hbm, char_hbm, out_hbm,
             char_v, idx_v, buf_v, zrows_v, sem_c, sem_g, sem_z, sem_o):
    cid = lax.axis_index("c")
    sid = lax.axis_index("s")
    b = sid
    delta = 1 - 2 * cid
    lane = lax.iota(jnp.int32, L)
    zeros = jnp.zeros((L,), dtype=jnp.float32)

    char_cp = pltpu.async_copy(char_hbm.at[b], char_v, sem_c)

    safe = jnp.full((L,), b * T, dtype=jnp.int32)
    for j in range(S // L):
        idx_v[pl.ds(j * L, L)] = safe

    @pl.loop(0, CHUNK)
    def _(r):
        for j in range(D // L):
            zrows_v[r, pl.ds(j * L, L)] = zeros

    def rows(c):
        return pl.multiple_of(c * CHUNK, CHUNK)

    def out_chunk(c):
        return out_hbm.at[b, pl.ds(rows(c), CHUNK), pl.ds(cid * D, D)]

    char_cp.wait()

    def scan_body(i, cnt_v):
        c = char_v[pl.ds(i * L, L)]
        p = i * L + lane
        q = p + delta
        m = (c == WB) & (q >= 0) & (q <= T - 1)
        slot = plsc.cumsum(m.astype(jnp.int32)) - 1 + cnt_v
        plsc.store_scatter(idx_v, [slot], q + b * T, mask=m)
        return cnt_v + plsc.all_reduce_population_count(m)

    cnt_v = lax.fori_loop(0, T // L, scan_body,
                          jnp.zeros((L,), jnp.int32), unroll=1)
    count = jnp.minimum(jnp.max(cnt_v), S)
    nch_g = (count + CHUNK - 1) // CHUNK
    ngc_g = (count + GCH - 1) // GCH

    def grows(c):
        return pl.multiple_of(c * GCH, GCH)

    @pl.loop(0, ngc_g)
    def _(c):
        pltpu.async_copy(rnn_hbm.at[idx_v.at[pl.ds(grows(c), GCH)]],
                         buf_v.at[pl.ds(grows(c), GCH)], sem_g)

    @pl.loop(nch_g, NCH)
    def _(c):
        pltpu.async_copy(zrows_v, out_chunk(c), sem_z)

    @pl.loop(0, ngc_g)
    def _(c):
        pltpu.make_async_copy(rnn_hbm.at[idx_v.at[pl.ds(grows(c), GCH)]],
                              buf_v.at[pl.ds(grows(c), GCH)], sem_g).wait()

    col_vecs = [j * L + lane for j in range(D // L)]
    rem = (CHUNK - count % CHUNK) % CHUNK

    @pl.loop(count, count + rem)
    def _(row):
        rvec = jnp.zeros((L,), dtype=jnp.int32) + row
        for cols in col_vecs:
            plsc.store_scatter(buf_v, [rvec, cols], zeros)

    @pl.loop(0, nch_g)
    def _(c):
        pltpu.async_copy(buf_v.at[pl.ds(rows(c), CHUNK)], out_chunk(c), sem_o)

    @pl.loop(nch_g, NCH)
    def _(c):
        pltpu.make_async_copy(zrows_v, out_chunk(c), sem_z).wait()

    @pl.loop(0, nch_g)
    def _(c):
        pltpu.make_async_copy(buf_v.at[pl.ds(rows(c), CHUNK)],
                              out_chunk(c), sem_o).wait()


def kernel(rnn_out, char_seq, mask):
    del mask
    rnn2 = rnn_out.reshape(B * T, D)
    char_seq = char_seq.astype(jnp.int32)

    sc_kernel = pl.kernel(
        _sc_body,
        out_type=jax.ShapeDtypeStruct((B, S, 2 * D), jnp.float32),
        mesh=plsc.VectorSubcoreMesh(
            core_axis_name="c", subcore_axis_name="s",
            num_cores=2, num_subcores=16,
        ),
        scratch_types=[
            pltpu.VMEM((T,), jnp.int32),
            pltpu.VMEM((T,), jnp.int32),
            pltpu.VMEM((S + CHUNK, D), jnp.float32),
            pltpu.VMEM((CHUNK, D), jnp.float32),
            pltpu.SemaphoreType.DMA,
            pltpu.SemaphoreType.DMA,
            pltpu.SemaphoreType.DMA,
            pltpu.SemaphoreType.DMA,
        ],
        compiler_params=pltpu.CompilerParams(needs_layout_passes=False),
    )
    return sc_kernel(rnn2, char_seq)


# speedup vs baseline: 1.0684x; 1.0684x over previous
import jax
import jax.numpy as jnp
from jax import lax
from jax.experimental import pallas as pl
from jax.experimental.pallas import tpu as pltpu
from jax.experimental.pallas import tpu_sc as plsc

B, T, D = 16, 2048, 256
S = 256
WB = 3
L = 16
CHUNK = 32
NCH = S // CHUNK
GCH = 8
NGC = S // GCH


def _sc_body(rnn_hbm, char_hbm, out_hbm,
             char_v, idx_v, buf_v, zrows_v, sem_c, sem_g, sem_z, sem_o):
    cid = lax.axis_index("c")
    sid = lax.axis_index("s")
    b = sid
    delta = 1 - 2 * cid
    lane = lax.iota(jnp.int32, L)
    zeros = jnp.zeros((L,), dtype=jnp.float32)

    char_cp = pltpu.async_copy(char_hbm.at[b], char_v, sem_c)

    safe = jnp.full((L,), b * T, dtype=jnp.int32)
    for j in range(S // L):
        idx_v[pl.ds(j * L, L)] = safe

    @pl.loop(0, CHUNK)
    def _(r):
        for j in range(D // L):
            zrows_v[r, pl.ds(j * L, L)] = zeros

    def rows(c):
        return pl.multiple_of(c * CHUNK, CHUNK)

    def out_chunk(c):
        return out_hbm.at[b, pl.ds(rows(c), CHUNK), pl.ds(cid * D, D)]

    for c in range(2, NCH):
        pltpu.async_copy(zrows_v, out_chunk(c), sem_z)

    char_cp.wait()

    def scan_body(i, cnt_v):
        c = char_v[pl.ds(i * L, L)]
        p = i * L + lane
        q = p + delta
        m = (c == WB) & (q >= 0) & (q <= T - 1)
        slot = plsc.cumsum(m.astype(jnp.int32)) - 1 + cnt_v
        plsc.store_scatter(idx_v, [slot], q + b * T, mask=m)
        return cnt_v + plsc.all_reduce_population_count(m)

    cnt_v = lax.fori_loop(0, T // L, scan_body,
                          jnp.zeros((L,), jnp.int32), unroll=2)
    count = jnp.minimum(jnp.max(cnt_v), S)
    nch_g = (count + CHUNK - 1) // CHUNK
    ngc_g = (count + GCH - 1) // GCH

    def grows(c):
        return pl.multiple_of(c * GCH, GCH)

    @pl.loop(0, ngc_g)
    def _(c):
        pltpu.async_copy(rnn_hbm.at[idx_v.at[pl.ds(grows(c), GCH)]],
                         buf_v.at[pl.ds(grows(c), GCH)], sem_g)

    @pl.loop(jnp.minimum(nch_g, 2), 2)
    def _(c):
        pltpu.async_copy(zrows_v, out_chunk(c), sem_z)

    @pl.loop(0, ngc_g)
    def _(c):
        pltpu.make_async_copy(rnn_hbm.at[idx_v.at[pl.ds(grows(c), GCH)]],
                              buf_v.at[pl.ds(grows(c), GCH)], sem_g).wait()

    col_vecs = [j * L + lane for j in range(D // L)]
    rem = (CHUNK - count % CHUNK) % CHUNK

    @pl.loop(count, count + rem)
    def _(row):
        rvec = jnp.zeros((L,), dtype=jnp.int32) + row
        for cols in col_vecs:
            plsc.store_scatter(buf_v, [rvec, cols], zeros)

    @pl.loop(0, nch_g)
    def _(c):
        pltpu.async_copy(buf_v.at[pl.ds(rows(c), CHUNK)], out_chunk(c), sem_o)

    @pl.loop(jnp.minimum(nch_g, 2), NCH)
    def _(c):
        pltpu.make_async_copy(zrows_v, out_chunk(c), sem_z).wait()

    @pl.loop(0, nch_g)
    def _(c):
        pltpu.make_async_copy(buf_v.at[pl.ds(rows(c), CHUNK)],
                              out_chunk(c), sem_o).wait()


def kernel(rnn_out, char_seq, mask):
    del mask
    rnn2 = rnn_out.reshape(B * T, D)
    char_seq = char_seq.astype(jnp.int32)

    sc_kernel = pl.kernel(
        _sc_body,
        out_type=jax.ShapeDtypeStruct((B, S, 2 * D), jnp.float32),
        mesh=plsc.VectorSubcoreMesh(
            core_axis_name="c", subcore_axis_name="s",
            num_cores=2, num_subcores=16,
        ),
        scratch_types=[
            pltpu.VMEM((T,), jnp.int32),
            pltpu.VMEM((T,), jnp.int32),
            pltpu.VMEM((S + CHUNK, D), jnp.float32),
            pltpu.VMEM((CHUNK, D), jnp.float32),
            pltpu.SemaphoreType.DMA,
            pltpu.SemaphoreType.DMA,
            pltpu.SemaphoreType.DMA,
            pltpu.SemaphoreType.DMA,
        ],
        compiler_params=pltpu.CompilerParams(needs_layout_passes=False),
    )
    return sc_kernel(rnn2, char_seq)
